# Initial kernel scaffold; baseline (speedup 1.0000x reference)
import jax
import jax.numpy as jnp
from jax.experimental import pallas as pl


def kernel(hidden_states, router_logits, w1, w3, w2):
    raise NotImplementedError("write your pallas kernel here")


# speedup vs baseline: 1.1136x; 1.1136x over previous
import functools

import jax
import jax.numpy as jnp
from jax import lax
from jax.experimental import pallas as pl
from jax.experimental.pallas import tpu as pltpu

E = 8
K = 2
H = 1024
I = 2816
T = 2048
BI = 256
NI = I // BI


def _coef_body(logits_ref, coef_ref):
    l = logits_ref[...]
    idx = lax.broadcasted_iota(jnp.int32, l.shape, 1)
    m1 = jnp.max(l, axis=-1, keepdims=True)
    i1 = jnp.min(jnp.where(l == m1, idx, E), axis=-1, keepdims=True)
    sel1 = idx == i1
    lmask = jnp.where(sel1, -jnp.inf, l)
    m2 = jnp.max(lmask, axis=-1, keepdims=True)
    i2 = jnp.min(jnp.where(lmask == m2, idx, E), axis=-1, keepdims=True)
    sel2 = idx == i2
    w1 = 1.0 / (1.0 + jnp.exp(m2 - m1))
    w2 = 1.0 - w1
    coef_ref[...] = jnp.where(sel1, w1, 0.0) + jnp.where(sel2, w2, 0.0)


def _moe_body(x_ref, w1_ref, w3_ref, w2_ref, coef_ref, out_ref):
    e = pl.program_id(0)
    i = pl.program_id(1)
    x = x_ref[...]
    g = lax.dot_general(x, w1_ref[0], (((1,), (1,)), ((), ())),
                        preferred_element_type=jnp.float32)
    u = lax.dot_general(x, w3_ref[0], (((1,), (1,)), ((), ())),
                        preferred_element_type=jnp.float32)
    h = (g * jax.nn.sigmoid(g) * u).astype(jnp.bfloat16)
    po = lax.dot_general(h, w2_ref[0], (((1,), (1,)), ((), ())),
                         preferred_element_type=jnp.float32)
    coef = coef_ref[...]
    eidx = lax.broadcasted_iota(jnp.int32, coef.shape, 1)
    ccol = jnp.sum(jnp.where(eidx == e, coef, 0.0), axis=-1, keepdims=True)
    contrib = ccol * po

    @pl.when(jnp.logical_and(e == 0, i == 0))
    def _():
        out_ref[...] = contrib

    @pl.when(jnp.logical_not(jnp.logical_and(e == 0, i == 0)))
    def _():
        out_ref[...] += contrib


def kernel(hidden_states, router_logits, w1, w3, w2):
    x = hidden_states.reshape(-1, H)
    coef = pl.pallas_call(
        _coef_body,
        out_shape=jax.ShapeDtypeStruct((T, E), jnp.float32),
    )(router_logits.astype(jnp.float32))

    xb = x.astype(jnp.bfloat16)
    w1b = w1.astype(jnp.bfloat16)
    w3b = w3.astype(jnp.bfloat16)
    w2b = w2.astype(jnp.bfloat16)

    out = pl.pallas_call(
        _moe_body,
        grid=(E, NI),
        in_specs=[
            pl.BlockSpec((T, H), lambda e, i: (0, 0)),
            pl.BlockSpec((1, BI, H), lambda e, i: (e, i, 0)),
            pl.BlockSpec((1, BI, H), lambda e, i: (e, i, 0)),
            pl.BlockSpec((1, H, BI), lambda e, i: (e, 0, i)),
            pl.BlockSpec((T, E), lambda e, i: (0, 0)),
        ],
        out_specs=pl.BlockSpec((T, H), lambda e, i: (0, 0)),
        out_shape=jax.ShapeDtypeStruct((T, H), jnp.float32),
    )(xb, w1b, w3b, w2b, coef)
    return out


# speedup vs baseline: 1.1733x; 1.0535x over previous
import functools

import jax
import jax.numpy as jnp
from jax import lax
from jax.experimental import pallas as pl
from jax.experimental.pallas import tpu as pltpu
from jax.experimental.pallas import tpu_sc as plsc

E = 8
H = 1024
I = 2816
T = 2048
BT = 256
NB = (4096 + E * BT) // BT
NROWS = NB * BT
BI = 256
NI = I // BI
NBPAD = 32

L = 16
NW = 32
TPW = T // NW
NG = T // L
GPW = TPW // L


def _routing_group(lg_v, j):
    le = [lg_v[e, pl.ds(j * L, L)] for e in range(E)]
    m1 = le[0]
    for e in range(1, E):
        m1 = jnp.maximum(m1, le[e])
    i1 = jnp.full((L,), E, jnp.int32)
    for e in range(E):
        i1 = jnp.minimum(i1, jnp.where(le[e] == m1, e, E))
    le2 = [jnp.where(i1 == e, -jnp.inf, le[e]) for e in range(E)]
    m2 = le2[0]
    for e in range(1, E):
        m2 = jnp.maximum(m2, le2[e])
    i2 = jnp.full((L,), E, jnp.int32)
    for e in range(E):
        i2 = jnp.minimum(i2, jnp.where(le2[e] == m2, e, E))
    wa = 1.0 / (1.0 + jnp.exp(m2 - m1))
    return i1, i2, wa


def _dispatch_body(lgT, x, xs, pos0, pos1, w0, w1, be,
                   lg_v, xc_v, e0_v, e1_v, w0_v, w1_v, p0_v, p1_v,
                   cnt_v, mybase_v, be_v, sem):
    wid = lax.axis_index("s") * 2 + lax.axis_index("c")
    base = wid * TPW
    lane = jnp.arange(L, dtype=jnp.int32)
    pltpu.sync_copy(lgT, lg_v)
    pltpu.sync_copy(x.at[pl.ds(base, TPW)], xc_v)

    cnt_v[...] = jnp.zeros((L,), jnp.int32)
    myfirst = wid * GPW
    ones = jnp.ones((L,), jnp.int32)

    def grp(j, _):
        @pl.when(j == myfirst)
        def _():
            mybase_v[...] = cnt_v[...]

        i1, i2, wa = _routing_group(lg_v, j)

        @pl.when(jnp.logical_and(j >= myfirst, j < myfirst + GPW))
        def _():
            loc = (j - myfirst) * L
            e0_v[pl.ds(loc, L)] = i1
            e1_v[pl.ds(loc, L)] = i2
            w0_v[pl.ds(loc, L)] = wa
            w1_v[pl.ds(loc, L)] = 1.0 - wa

        plsc.addupdate_scatter(cnt_v, [i1], ones)
        plsc.addupdate_scatter(cnt_v, [i2], ones)
        return 0

    lax.fori_loop(0, NG, grp, 0)

    totals = cnt_v[...]
    my_base = mybase_v[...]
    pt = ((totals + (BT - 1)) // BT) * BT
    seg_start = plsc.cumsum(pt) - pt
    offs_vec = seg_start + my_base
    offs = [jnp.sum(jnp.where(lane == e, offs_vec, 0)) for e in range(E)]
    seg_sc = [jnp.sum(jnp.where(lane == e, seg_start, 0)) for e in range(E)]

    for j in range(GPW):
        for ev_ref, pv in ((e0_v, p0_v), (e1_v, p1_v)):
            ev = ev_ref[pl.ds(j * L, L)]
            posv = jnp.zeros((L,), jnp.int32)
            for e in range(E):
                m = ev == e
                mi = m.astype(jnp.int32)
                incl = plsc.cumsum(mi)
                posv = jnp.where(m, offs[e] + incl - 1, posv)
                offs[e] = offs[e] + jnp.sum(mi)
            pv[pl.ds(j * L, L)] = posv

    pltpu.sync_copy(p0_v, pos0.at[pl.ds(base, TPW)])
    pltpu.sync_copy(p1_v, pos1.at[pl.ds(base, TPW)])
    pltpu.sync_copy(w0_v, w0.at[pl.ds(base, TPW)])
    pltpu.sync_copy(w1_v, w1.at[pl.ds(base, TPW)])

    pltpu.async_copy(xc_v, xs.at[p0_v], sem).wait()
    pltpu.async_copy(xc_v, xs.at[p1_v], sem).wait()

    @pl.when(wid == 0)
    def _():
        for g in range(NBPAD // L):
            rowstart = (jnp.arange(L, dtype=jnp.int32) + g * L) * BT
            bev = jnp.zeros((L,), jnp.int32)
            for e in range(E):
                bev = bev + (rowstart >= seg_sc[e]).astype(jnp.int32)
            be_v[pl.ds(g * L, L)] = jnp.clip(bev - 1, 0, E - 1)
        pltpu.sync_copy(be_v, be)


def _dispatch(logitsT, x):
    mesh = plsc.VectorSubcoreMesh(core_axis_name="c", subcore_axis_name="s",
                                  num_cores=2, num_subcores=16)
    return pl.kernel(
        _dispatch_body,
        out_type=(
            jax.ShapeDtypeStruct((NROWS, H), jnp.float32),
            jax.ShapeDtypeStruct((T,), jnp.int32),
            jax.ShapeDtypeStruct((T,), jnp.int32),
            jax.ShapeDtypeStruct((T,), jnp.float32),
            jax.ShapeDtypeStruct((T,), jnp.float32),
            jax.ShapeDtypeStruct((NBPAD,), jnp.int32),
        ),
        mesh=mesh,
        compiler_params=pltpu.CompilerParams(needs_layout_passes=False),
        scratch_types=(
            pltpu.VMEM((E, T), jnp.float32),
            pltpu.VMEM((TPW, H), jnp.float32),
            pltpu.VMEM((TPW,), jnp.int32),
            pltpu.VMEM((TPW,), jnp.int32),
            pltpu.VMEM((TPW,), jnp.float32),
            pltpu.VMEM((TPW,), jnp.float32),
            pltpu.VMEM((TPW,), jnp.int32),
            pltpu.VMEM((TPW,), jnp.int32),
            pltpu.VMEM((L,), jnp.int32),
            pltpu.VMEM((L,), jnp.int32),
            pltpu.VMEM((NBPAD,), jnp.int32),
            pltpu.SemaphoreType.DMA,
        ),
    )(logitsT, x)


def _gmm_body(be_ref, xs_ref, w1_ref, w3_ref, w2_ref, y_ref):
    i = pl.program_id(1)
    xb = xs_ref[...].astype(jnp.bfloat16)
    g = lax.dot_general(xb, w1_ref[0], (((1,), (1,)), ((), ())),
                        preferred_element_type=jnp.float32)
    u = lax.dot_general(xb, w3_ref[0], (((1,), (1,)), ((), ())),
                        preferred_element_type=jnp.float32)
    h = (g * jax.nn.sigmoid(g) * u).astype(jnp.bfloat16)
    po = lax.dot_general(h, w2_ref[0], (((1,), (1,)), ((), ())),
                         preferred_element_type=jnp.float32)

    @pl.when(i == 0)
    def _():
        y_ref[...] = po

    @pl.when(i != 0)
    def _():
        y_ref[...] += po


def _gmm(be, xs, w1b, w3b, w2b):
    grid_spec = pltpu.PrefetchScalarGridSpec(
        num_scalar_prefetch=1,
        grid=(NB, NI),
        in_specs=[
            pl.BlockSpec((BT, H), lambda b, i, be_ref: (b, 0)),
            pl.BlockSpec((1, BI, H), lambda b, i, be_ref: (be_ref[b], i, 0)),
            pl.BlockSpec((1, BI, H), lambda b, i, be_ref: (be_ref[b], i, 0)),
            pl.BlockSpec((1, H, BI), lambda b, i, be_ref: (be_ref[b], 0, i)),
        ],
        out_specs=pl.BlockSpec((BT, H), lambda b, i, be_ref: (b, 0)),
    )
    return pl.pallas_call(
        _gmm_body,
        grid_spec=grid_spec,
        out_shape=jax.ShapeDtypeStruct((NROWS, H), jnp.float32),
    )(be, xs, w1b, w3b, w2b)


def _combine_body(y, pos0, pos1, w0, w1, out,
                  i0_v, i1_v, wv0, wv1, r0_v, r1_v, ob_v, sem):
    wid = lax.axis_index("s") * 2 + lax.axis_index("c")
    for c in range(TPW // L):
        base = wid * TPW + c * L
        pltpu.sync_copy(pos0.at[pl.ds(base, L)], i0_v)
        pltpu.sync_copy(pos1.at[pl.ds(base, L)], i1_v)
        pltpu.sync_copy(w0.at[pl.ds(base, L)], wv0)
        pltpu.sync_copy(w1.at[pl.ds(base, L)], wv1)
        pltpu.async_copy(y.at[i0_v], r0_v, sem).wait()
        pltpu.async_copy(y.at[i1_v], r1_v, sem).wait()

        def row(j, _):
            ws0 = plsc.load_gather(wv0, [jnp.full((L,), j, jnp.int32)])
            ws1 = plsc.load_gather(wv1, [jnp.full((L,), j, jnp.int32)])
            for q in range(H // L):
                s = pl.ds(q * L, L)
                ob_v[j, s] = ws0 * r0_v[j, s] + ws1 * r1_v[j, s]
            return 0

        lax.fori_loop(0, L, row, 0)
        pltpu.sync_copy(ob_v, out.at[pl.ds(base, L)])


def _combine(y, pos0, pos1, w0, w1):
    mesh = plsc.VectorSubcoreMesh(core_axis_name="c", subcore_axis_name="s",
                                  num_cores=2, num_subcores=16)
    return pl.kernel(
        _combine_body,
        out_type=jax.ShapeDtypeStruct((T, H), jnp.float32),
        mesh=mesh,
        compiler_params=pltpu.CompilerParams(needs_layout_passes=False),
        scratch_types=(
            pltpu.VMEM((L,), jnp.int32),
            pltpu.VMEM((L,), jnp.int32),
            pltpu.VMEM((L,), jnp.float32),
            pltpu.VMEM((L,), jnp.float32),
            pltpu.VMEM((L, H), jnp.float32),
            pltpu.VMEM((L, H), jnp.float32),
            pltpu.VMEM((L, H), jnp.float32),
            pltpu.SemaphoreType.DMA,
        ),
    )(y, pos0, pos1, w0, w1)


def kernel(hidden_states, router_logits, w1, w3, w2):
    x = hidden_states.reshape(-1, H).astype(jnp.float32)
    logitsT = router_logits.astype(jnp.float32).T
    xs, pos0, pos1, wt0, wt1, be = _dispatch(logitsT, x)
    y = _gmm(be, xs,
             w1.astype(jnp.bfloat16), w3.astype(jnp.bfloat16),
             w2.astype(jnp.bfloat16))
    return _combine(y, pos0, pos1, wt0, wt1)


# speedup vs baseline: 1.7118x; 1.4590x over previous
import functools

import jax
import jax.numpy as jnp
from jax import lax
from jax.experimental import pallas as pl
from jax.experimental.pallas import tpu as pltpu
from jax.experimental.pallas import tpu_sc as plsc

E = 8
H = 1024
I = 2816
T = 2048
BT = 256
NB = (4096 + E * BT) // BT
NROWS = NB * BT
BI = 256
NI = I // BI
NBPAD = 32

L = 16
NW = 32
TPW = T // NW
NG = T // L
GPW = TPW // L


def _routing_group(lg_v, j):
    le = [lg_v[e, pl.ds(j * L, L)] for e in range(E)]
    m1 = le[0]
    for e in range(1, E):
        m1 = jnp.maximum(m1, le[e])
    i1 = jnp.full((L,), E, jnp.int32)
    for e in range(E):
        i1 = jnp.minimum(i1, jnp.where(le[e] == m1, e, E))
    le2 = [jnp.where(i1 == e, -jnp.inf, le[e]) for e in range(E)]
    m2 = le2[0]
    for e in range(1, E):
        m2 = jnp.maximum(m2, le2[e])
    i2 = jnp.full((L,), E, jnp.int32)
    for e in range(E):
        i2 = jnp.minimum(i2, jnp.where(le2[e] == m2, e, E))
    wa = 1.0 / (1.0 + jnp.exp(m2 - m1))
    return i1, i2, wa


def _dispatch_body(lgT, x, xs, pos0, pos1, w0, w1, be,
                   lg_v, xc_v, e0_v, e1_v, w0_v, w1_v, p0_v, p1_v,
                   cnt_v, mybase_v, be_v, sem):
    wid = lax.axis_index("s") * 2 + lax.axis_index("c")
    base = wid * TPW
    lane = jnp.arange(L, dtype=jnp.int32)
    pltpu.sync_copy(lgT, lg_v)
    pltpu.sync_copy(x.at[pl.ds(base, TPW)], xc_v)

    cnt_v[...] = jnp.zeros((L,), jnp.int32)
    myfirst = wid * GPW
    ones = jnp.ones((L,), jnp.int32)

    def grp(j, _):
        @pl.when(j == myfirst)
        def _():
            mybase_v[...] = cnt_v[...]

        i1, i2, wa = _routing_group(lg_v, j)

        @pl.when(jnp.logical_and(j >= myfirst, j < myfirst + GPW))
        def _():
            loc = (j - myfirst) * L
            e0_v[pl.ds(loc, L)] = i1
            e1_v[pl.ds(loc, L)] = i2
            w0_v[pl.ds(loc, L)] = wa
            w1_v[pl.ds(loc, L)] = 1.0 - wa

        plsc.addupdate_scatter(cnt_v, [i1], ones)
        plsc.addupdate_scatter(cnt_v, [i2], ones)
        return 0

    lax.fori_loop(0, NG, grp, 0)

    totals = cnt_v[...]
    my_base = mybase_v[...]
    pt = ((totals + (BT - 1)) // BT) * BT
    seg_start = plsc.cumsum(pt) - pt
    offs_vec = seg_start + my_base
    offs = [jnp.sum(jnp.where(lane == e, offs_vec, 0)) for e in range(E)]
    seg_sc = [jnp.sum(jnp.where(lane == e, seg_start, 0)) for e in range(E)]

    for j in range(GPW):
        for ev_ref, pv in ((e0_v, p0_v), (e1_v, p1_v)):
            ev = ev_ref[pl.ds(j * L, L)]
            posv = jnp.zeros((L,), jnp.int32)
            for e in range(E):
                m = ev == e
                mi = m.astype(jnp.int32)
                incl = plsc.cumsum(mi)
                posv = jnp.where(m, offs[e] + incl - 1, posv)
                offs[e] = offs[e] + jnp.sum(mi)
            pv[pl.ds(j * L, L)] = posv

    pltpu.sync_copy(p0_v, pos0.at[pl.ds(base, TPW)])
    pltpu.sync_copy(p1_v, pos1.at[pl.ds(base, TPW)])
    pltpu.sync_copy(w0_v, w0.at[pl.ds(base, TPW)])
    pltpu.sync_copy(w1_v, w1.at[pl.ds(base, TPW)])

    pltpu.async_copy(xc_v, xs.at[p0_v], sem).wait()
    pltpu.async_copy(xc_v, xs.at[p1_v], sem).wait()

    @pl.when(wid == 0)
    def _():
        for g in range(NBPAD // L):
            rowstart = (jnp.arange(L, dtype=jnp.int32) + g * L) * BT
            bev = jnp.zeros((L,), jnp.int32)
            for e in range(E):
                bev = bev + (rowstart >= seg_sc[e]).astype(jnp.int32)
            be_v[pl.ds(g * L, L)] = jnp.clip(bev - 1, 0, E - 1)
        pltpu.sync_copy(be_v, be)


def _dispatch(logitsT, x):
    mesh = plsc.VectorSubcoreMesh(core_axis_name="c", subcore_axis_name="s",
                                  num_cores=2, num_subcores=16)
    return pl.kernel(
        _dispatch_body,
        out_type=(
            jax.ShapeDtypeStruct((NROWS, H), jnp.float32),
            jax.ShapeDtypeStruct((T,), jnp.int32),
            jax.ShapeDtypeStruct((T,), jnp.int32),
            jax.ShapeDtypeStruct((T,), jnp.float32),
            jax.ShapeDtypeStruct((T,), jnp.float32),
            jax.ShapeDtypeStruct((NBPAD,), jnp.int32),
        ),
        mesh=mesh,
        compiler_params=pltpu.CompilerParams(needs_layout_passes=False),
        scratch_types=(
            pltpu.VMEM((E, T), jnp.float32),
            pltpu.VMEM((TPW, H), jnp.float32),
            pltpu.VMEM((TPW,), jnp.int32),
            pltpu.VMEM((TPW,), jnp.int32),
            pltpu.VMEM((TPW,), jnp.float32),
            pltpu.VMEM((TPW,), jnp.float32),
            pltpu.VMEM((TPW,), jnp.int32),
            pltpu.VMEM((TPW,), jnp.int32),
            pltpu.VMEM((L,), jnp.int32),
            pltpu.VMEM((L,), jnp.int32),
            pltpu.VMEM((NBPAD,), jnp.int32),
            pltpu.SemaphoreType.DMA,
        ),
    )(logitsT, x)


def _gmm_body(be_ref, xs_ref, w1_ref, w3_ref, w2_ref, y_ref):
    xb = xs_ref[...].astype(jnp.bfloat16)
    for i in range(NI):
        w1c = w1_ref[0, pl.ds(i * BI, BI), :]
        w3c = w3_ref[0, pl.ds(i * BI, BI), :]
        w2c = w2_ref[0, :, pl.ds(i * BI, BI)]
        g = lax.dot_general(xb, w1c, (((1,), (1,)), ((), ())),
                            preferred_element_type=jnp.float32)
        u = lax.dot_general(xb, w3c, (((1,), (1,)), ((), ())),
                            preferred_element_type=jnp.float32)
        h = (g * jax.nn.sigmoid(g) * u).astype(jnp.bfloat16)
        po = lax.dot_general(h, w2c, (((1,), (1,)), ((), ())),
                             preferred_element_type=jnp.float32)
        if i == 0:
            y_ref[...] = po
        else:
            y_ref[...] += po


def _gmm(be, xs, w1b, w3b, w2b):
    grid_spec = pltpu.PrefetchScalarGridSpec(
        num_scalar_prefetch=1,
        grid=(NB,),
        in_specs=[
            pl.BlockSpec((BT, H), lambda b, be_ref: (b, 0)),
            pl.BlockSpec((1, I, H), lambda b, be_ref: (be_ref[b], 0, 0)),
            pl.BlockSpec((1, I, H), lambda b, be_ref: (be_ref[b], 0, 0)),
            pl.BlockSpec((1, H, I), lambda b, be_ref: (be_ref[b], 0, 0)),
        ],
        out_specs=pl.BlockSpec((BT, H), lambda b, be_ref: (b, 0)),
    )
    return pl.pallas_call(
        _gmm_body,
        grid_spec=grid_spec,
        out_shape=jax.ShapeDtypeStruct((NROWS, H), jnp.float32),
    )(be, xs, w1b, w3b, w2b)


def _combine_body(y, pos0, pos1, w0, w1, out,
                  i0_v, i1_v, wv0, wv1, r0_v, r1_v, ob_v, sem):
    wid = lax.axis_index("s") * 2 + lax.axis_index("c")
    for c in range(TPW // L):
        base = wid * TPW + c * L
        pltpu.sync_copy(pos0.at[pl.ds(base, L)], i0_v)
        pltpu.sync_copy(pos1.at[pl.ds(base, L)], i1_v)
        pltpu.sync_copy(w0.at[pl.ds(base, L)], wv0)
        pltpu.sync_copy(w1.at[pl.ds(base, L)], wv1)
        pltpu.async_copy(y.at[i0_v], r0_v, sem).wait()
        pltpu.async_copy(y.at[i1_v], r1_v, sem).wait()

        def row(j, _):
            ws0 = plsc.load_gather(wv0, [jnp.full((L,), j, jnp.int32)])
            ws1 = plsc.load_gather(wv1, [jnp.full((L,), j, jnp.int32)])
            for q in range(H // L):
                s = pl.ds(q * L, L)
                ob_v[j, s] = ws0 * r0_v[j, s] + ws1 * r1_v[j, s]
            return 0

        lax.fori_loop(0, L, row, 0)
        pltpu.sync_copy(ob_v, out.at[pl.ds(base, L)])


def _combine(y, pos0, pos1, w0, w1):
    mesh = plsc.VectorSubcoreMesh(core_axis_name="c", subcore_axis_name="s",
                                  num_cores=2, num_subcores=16)
    return pl.kernel(
        _combine_body,
        out_type=jax.ShapeDtypeStruct((T, H), jnp.float32),
        mesh=mesh,
        compiler_params=pltpu.CompilerParams(needs_layout_passes=False),
        scratch_types=(
            pltpu.VMEM((L,), jnp.int32),
            pltpu.VMEM((L,), jnp.int32),
            pltpu.VMEM((L,), jnp.float32),
            pltpu.VMEM((L,), jnp.float32),
            pltpu.VMEM((L, H), jnp.float32),
            pltpu.VMEM((L, H), jnp.float32),
            pltpu.VMEM((L, H), jnp.float32),
            pltpu.SemaphoreType.DMA,
        ),
    )(y, pos0, pos1, w0, w1)


def kernel(hidden_states, router_logits, w1, w3, w2):
    x = hidden_states.reshape(-1, H).astype(jnp.float32)
    logitsT = router_logits.astype(jnp.float32).T
    xs, pos0, pos1, wt0, wt1, be = _dispatch(logitsT, x)
    y = _gmm(be, xs,
             w1.astype(jnp.bfloat16), w3.astype(jnp.bfloat16),
             w2.astype(jnp.bfloat16))
    return _combine(y, pos0, pos1, wt0, wt1)


# speedup vs baseline: 1.8366x; 1.0729x over previous
import functools

import jax
import jax.numpy as jnp
from jax import lax
from jax.experimental import pallas as pl
from jax.experimental.pallas import tpu as pltpu
from jax.experimental.pallas import tpu_sc as plsc

E = 8
H = 1024
I = 2816
T = 2048
BT = 256
NB = (4096 + E * BT) // BT
NROWS = NB * BT
BI = 256
NI = I // BI
NBPAD = 32

L = 16
NW = 32
TPW = T // NW
NG = T // L
GPW = TPW // L


def _routing_group(lg_v, j):
    le = [lg_v[e, pl.ds(j * L, L)] for e in range(E)]
    m1 = le[0]
    for e in range(1, E):
        m1 = jnp.maximum(m1, le[e])
    i1 = jnp.full((L,), E, jnp.int32)
    for e in range(E):
        i1 = jnp.minimum(i1, jnp.where(le[e] == m1, e, E))
    le2 = [jnp.where(i1 == e, -jnp.inf, le[e]) for e in range(E)]
    m2 = le2[0]
    for e in range(1, E):
        m2 = jnp.maximum(m2, le2[e])
    i2 = jnp.full((L,), E, jnp.int32)
    for e in range(E):
        i2 = jnp.minimum(i2, jnp.where(le2[e] == m2, e, E))
    wa = 1.0 / (1.0 + jnp.exp(m2 - m1))
    return i1, i2, wa


def _dispatch_body(lgT, x, xs, pos0, pos1, w0, w1, be,
                   lg_v, xc_v, e0_v, e1_v, w0_v, w1_v, p0_v, p1_v,
                   cnt_v, mybase_v, be_v, sem):
    wid = lax.axis_index("s") * 2 + lax.axis_index("c")
    base = wid * TPW
    lane = jnp.arange(L, dtype=jnp.int32)
    pltpu.sync_copy(lgT, lg_v)
    pltpu.sync_copy(x.at[pl.ds(base, TPW)], xc_v)

    cnt_v[...] = jnp.zeros((L,), jnp.int32)
    myfirst = wid * GPW
    ones = jnp.ones((L,), jnp.int32)

    def grp(j, _):
        @pl.when(j == myfirst)
        def _():
            mybase_v[...] = cnt_v[...]

        i1, i2, wa = _routing_group(lg_v, j)

        @pl.when(jnp.logical_and(j >= myfirst, j < myfirst + GPW))
        def _():
            loc = (j - myfirst) * L
            e0_v[pl.ds(loc, L)] = i1
            e1_v[pl.ds(loc, L)] = i2
            w0_v[pl.ds(loc, L)] = wa
            w1_v[pl.ds(loc, L)] = 1.0 - wa

        plsc.addupdate_scatter(cnt_v, [i1], ones)
        plsc.addupdate_scatter(cnt_v, [i2], ones)
        return 0

    lax.fori_loop(0, NG, grp, 0)

    totals = cnt_v[...]
    my_base = mybase_v[...]
    pt = ((totals + (BT - 1)) // BT) * BT
    seg_start = plsc.cumsum(pt) - pt
    offs_vec = seg_start + my_base
    offs = [jnp.sum(jnp.where(lane == e, offs_vec, 0)) for e in range(E)]
    seg_sc = [jnp.sum(jnp.where(lane == e, seg_start, 0)) for e in range(E)]

    for j in range(GPW):
        for ev_ref, pv in ((e0_v, p0_v), (e1_v, p1_v)):
            ev = ev_ref[pl.ds(j * L, L)]
            posv = jnp.zeros((L,), jnp.int32)
            for e in range(E):
                m = ev == e
                mi = m.astype(jnp.int32)
                incl = plsc.cumsum(mi)
                posv = jnp.where(m, offs[e] + incl - 1, posv)
                offs[e] = offs[e] + jnp.sum(mi)
            pv[pl.ds(j * L, L)] = posv

    pltpu.sync_copy(p0_v, pos0.at[pl.ds(base, TPW)])
    pltpu.sync_copy(p1_v, pos1.at[pl.ds(base, TPW)])
    pltpu.sync_copy(w0_v, w0.at[pl.ds(base, TPW)])
    pltpu.sync_copy(w1_v, w1.at[pl.ds(base, TPW)])

    cp0 = pltpu.async_copy(xc_v, xs.at[p0_v], sem)
    cp1 = pltpu.async_copy(xc_v, xs.at[p1_v], sem)
    cp0.wait()
    cp1.wait()

    @pl.when(wid == 0)
    def _():
        used = jnp.sum(pt) // BT
        for g in range(NBPAD // L):
            gidx = jnp.arange(L, dtype=jnp.int32) + g * L
            rowstart = gidx * BT
            bev = jnp.zeros((L,), jnp.int32)
            for e in range(E):
                bev = bev + (rowstart >= seg_sc[e]).astype(jnp.int32)
            bev = jnp.clip(bev - 1, 0, E - 1)
            be_v[pl.ds(g * L, L)] = jnp.where(gidx == NB, used, bev)
        pltpu.sync_copy(be_v, be)


def _dispatch(logitsT, x):
    mesh = plsc.VectorSubcoreMesh(core_axis_name="c", subcore_axis_name="s",
                                  num_cores=2, num_subcores=16)
    return pl.kernel(
        _dispatch_body,
        out_type=(
            jax.ShapeDtypeStruct((NROWS, H), jnp.float32),
            jax.ShapeDtypeStruct((T,), jnp.int32),
            jax.ShapeDtypeStruct((T,), jnp.int32),
            jax.ShapeDtypeStruct((T,), jnp.float32),
            jax.ShapeDtypeStruct((T,), jnp.float32),
            jax.ShapeDtypeStruct((NBPAD,), jnp.int32),
        ),
        mesh=mesh,
        compiler_params=pltpu.CompilerParams(needs_layout_passes=False),
        scratch_types=(
            pltpu.VMEM((E, T), jnp.float32),
            pltpu.VMEM((TPW, H), jnp.float32),
            pltpu.VMEM((TPW,), jnp.int32),
            pltpu.VMEM((TPW,), jnp.int32),
            pltpu.VMEM((TPW,), jnp.float32),
            pltpu.VMEM((TPW,), jnp.float32),
            pltpu.VMEM((TPW,), jnp.int32),
            pltpu.VMEM((TPW,), jnp.int32),
            pltpu.VMEM((L,), jnp.int32),
            pltpu.VMEM((L,), jnp.int32),
            pltpu.VMEM((NBPAD,), jnp.int32),
            pltpu.SemaphoreType.DMA,
        ),
    )(logitsT, x)


def _gmm_body(be_ref, xs_ref, w1_ref, w3_ref, w2_ref, y_ref):
    b = pl.program_id(0)

    @pl.when(b < be_ref[NB])
    def _():
        _gmm_compute(xs_ref, w1_ref, w3_ref, w2_ref, y_ref)


def _gmm_compute(xs_ref, w1_ref, w3_ref, w2_ref, y_ref):
    xb = xs_ref[...].astype(jnp.bfloat16)
    for i in range(NI):
        w1c = w1_ref[0, pl.ds(i * BI, BI), :]
        w3c = w3_ref[0, pl.ds(i * BI, BI), :]
        w2c = w2_ref[0, :, pl.ds(i * BI, BI)]
        g = lax.dot_general(xb, w1c, (((1,), (1,)), ((), ())),
                            preferred_element_type=jnp.float32)
        u = lax.dot_general(xb, w3c, (((1,), (1,)), ((), ())),
                            preferred_element_type=jnp.float32)
        h = (g * jax.nn.sigmoid(g) * u).astype(jnp.bfloat16)
        po = lax.dot_general(h, w2c, (((1,), (1,)), ((), ())),
                             preferred_element_type=jnp.float32)
        if i == 0:
            y_ref[...] = po
        else:
            y_ref[...] += po


def _gmm(be, xs, w1b, w3b, w2b):
    grid_spec = pltpu.PrefetchScalarGridSpec(
        num_scalar_prefetch=1,
        grid=(NB,),
        in_specs=[
            pl.BlockSpec((BT, H), lambda b, be_ref: (b, 0)),
            pl.BlockSpec((1, I, H), lambda b, be_ref: (be_ref[b], 0, 0)),
            pl.BlockSpec((1, I, H), lambda b, be_ref: (be_ref[b], 0, 0)),
            pl.BlockSpec((1, H, I), lambda b, be_ref: (be_ref[b], 0, 0)),
        ],
        out_specs=pl.BlockSpec((BT, H), lambda b, be_ref: (b, 0)),
    )
    return pl.pallas_call(
        _gmm_body,
        grid_spec=grid_spec,
        out_shape=jax.ShapeDtypeStruct((NROWS, H), jnp.float32),
    )(be, xs, w1b, w3b, w2b)


def _combine_body(y, pos0, pos1, w0, w1, out,
                  i0_v, i1_v, wv0, wv1, r0_v, r1_v, ob_v, sem):
    wid = lax.axis_index("s") * 2 + lax.axis_index("c")
    for c in range(TPW // L):
        base = wid * TPW + c * L
        pltpu.sync_copy(pos0.at[pl.ds(base, L)], i0_v)
        pltpu.sync_copy(pos1.at[pl.ds(base, L)], i1_v)
        pltpu.sync_copy(w0.at[pl.ds(base, L)], wv0)
        pltpu.sync_copy(w1.at[pl.ds(base, L)], wv1)
        g0 = pltpu.async_copy(y.at[i0_v], r0_v, sem)
        g1 = pltpu.async_copy(y.at[i1_v], r1_v, sem)
        g0.wait()
        g1.wait()

        def row(j, _):
            ws0 = plsc.load_gather(wv0, [jnp.full((L,), j, jnp.int32)])
            ws1 = plsc.load_gather(wv1, [jnp.full((L,), j, jnp.int32)])
            for q in range(H // L):
                s = pl.ds(q * L, L)
                ob_v[j, s] = ws0 * r0_v[j, s] + ws1 * r1_v[j, s]
            return 0

        lax.fori_loop(0, L, row, 0)
        pltpu.sync_copy(ob_v, out.at[pl.ds(base, L)])


def _combine(y, pos0, pos1, w0, w1):
    mesh = plsc.VectorSubcoreMesh(core_axis_name="c", subcore_axis_name="s",
                                  num_cores=2, num_subcores=16)
    return pl.kernel(
        _combine_body,
        out_type=jax.ShapeDtypeStruct((T, H), jnp.float32),
        mesh=mesh,
        compiler_params=pltpu.CompilerParams(needs_layout_passes=False),
        scratch_types=(
            pltpu.VMEM((L,), jnp.int32),
            pltpu.VMEM((L,), jnp.int32),
            pltpu.VMEM((L,), jnp.float32),
            pltpu.VMEM((L,), jnp.float32),
            pltpu.VMEM((L, H), jnp.float32),
            pltpu.VMEM((L, H), jnp.float32),
            pltpu.VMEM((L, H), jnp.float32),
            pltpu.SemaphoreType.DMA,
        ),
    )(y, pos0, pos1, w0, w1)


def kernel(hidden_states, router_logits, w1, w3, w2):
    x = hidden_states.reshape(-1, H).astype(jnp.float32)
    logitsT = router_logits.astype(jnp.float32).T
    xs, pos0, pos1, wt0, wt1, be = _dispatch(logitsT, x)
    y = _gmm(be, xs,
             w1.astype(jnp.bfloat16), w3.astype(jnp.bfloat16),
             w2.astype(jnp.bfloat16))
    return _combine(y, pos0, pos1, wt0, wt1)


# speedup vs baseline: 1.9432x; 1.0580x over previous
import functools

import jax
import jax.numpy as jnp
from jax import lax
from jax.experimental import pallas as pl
from jax.experimental.pallas import tpu as pltpu
from jax.experimental.pallas import tpu_sc as plsc

E = 8
H = 1024
I = 2816
T = 2048
BT = 256
NB = (4096 + E * BT) // BT
NROWS = NB * BT
BI = 256
NI = I // BI
NBPAD = 32

L = 16
NW = 32
TPW = T // NW
NG = T // L
GPW = TPW // L


def _routing_group(lg_v, j):
    le = [lg_v[e, pl.ds(j * L, L)] for e in range(E)]
    m1 = le[0]
    for e in range(1, E):
        m1 = jnp.maximum(m1, le[e])
    i1 = jnp.full((L,), E, jnp.int32)
    for e in range(E):
        i1 = jnp.minimum(i1, jnp.where(le[e] == m1, e, E))
    le2 = [jnp.where(i1 == e, -jnp.inf, le[e]) for e in range(E)]
    m2 = le2[0]
    for e in range(1, E):
        m2 = jnp.maximum(m2, le2[e])
    i2 = jnp.full((L,), E, jnp.int32)
    for e in range(E):
        i2 = jnp.minimum(i2, jnp.where(le2[e] == m2, e, E))
    wa = 1.0 / (1.0 + jnp.exp(m2 - m1))
    return i1, i2, wa


def _dispatch_body(lgT, x, xs, pos0, pos1, w0, w1, be,
                   lg_v, xc_v, e0_v, e1_v, w0_v, w1_v, p0_v, p1_v,
                   cnt_v, mybase_v, be_v, sem):
    wid = lax.axis_index("s") * 2 + lax.axis_index("c")
    base = wid * TPW
    lane = jnp.arange(L, dtype=jnp.int32)
    pltpu.sync_copy(lgT, lg_v)
    pltpu.sync_copy(x.at[pl.ds(base, TPW)], xc_v)

    cnt_v[...] = jnp.zeros((L,), jnp.int32)
    myfirst = wid * GPW
    ones = jnp.ones((L,), jnp.int32)

    def grp(j, _):
        @pl.when(j == myfirst)
        def _():
            mybase_v[...] = cnt_v[...]

        i1, i2, wa = _routing_group(lg_v, j)

        @pl.when(jnp.logical_and(j >= myfirst, j < myfirst + GPW))
        def _():
            loc = (j - myfirst) * L
            e0_v[pl.ds(loc, L)] = i1
            e1_v[pl.ds(loc, L)] = i2
            w0_v[pl.ds(loc, L)] = wa
            w1_v[pl.ds(loc, L)] = 1.0 - wa

        plsc.addupdate_scatter(cnt_v, [i1], ones)
        plsc.addupdate_scatter(cnt_v, [i2], ones)
        return 0

    lax.fori_loop(0, NG, grp, 0)

    totals = cnt_v[...]
    my_base = mybase_v[...]
    pt = ((totals + (BT - 1)) // BT) * BT
    seg_start = plsc.cumsum(pt) - pt
    offs_vec = seg_start + my_base
    offs = [jnp.sum(jnp.where(lane == e, offs_vec, 0)) for e in range(E)]
    seg_sc = [jnp.sum(jnp.where(lane == e, seg_start, 0)) for e in range(E)]

    for j in range(GPW):
        for ev_ref, pv in ((e0_v, p0_v), (e1_v, p1_v)):
            ev = ev_ref[pl.ds(j * L, L)]
            posv = jnp.zeros((L,), jnp.int32)
            for e in range(E):
                m = ev == e
                mi = m.astype(jnp.int32)
                incl = plsc.cumsum(mi)
                posv = jnp.where(m, offs[e] + incl - 1, posv)
                offs[e] = offs[e] + jnp.sum(mi)
            pv[pl.ds(j * L, L)] = posv

    pltpu.sync_copy(p0_v, pos0.at[pl.ds(base, TPW)])
    pltpu.sync_copy(p1_v, pos1.at[pl.ds(base, TPW)])
    pltpu.sync_copy(w0_v, w0.at[pl.ds(base, TPW)])
    pltpu.sync_copy(w1_v, w1.at[pl.ds(base, TPW)])

    cp0 = pltpu.async_copy(xc_v, xs.at[p0_v], sem)
    cp1 = pltpu.async_copy(xc_v, xs.at[p1_v], sem)
    cp0.wait()
    cp1.wait()

    @pl.when(wid == 0)
    def _():
        used = jnp.sum(pt) // BT
        for g in range(NBPAD // L):
            gidx = jnp.arange(L, dtype=jnp.int32) + g * L
            rowstart = gidx * BT
            bev = jnp.zeros((L,), jnp.int32)
            for e in range(E):
                bev = bev + (rowstart >= seg_sc[e]).astype(jnp.int32)
            bev = jnp.clip(bev - 1, 0, E - 1)
            be_v[pl.ds(g * L, L)] = jnp.where(gidx == NB, used, bev)
        pltpu.sync_copy(be_v, be)


def _dispatch(logitsT, x):
    mesh = plsc.VectorSubcoreMesh(core_axis_name="c", subcore_axis_name="s",
                                  num_cores=2, num_subcores=16)
    return pl.kernel(
        _dispatch_body,
        out_type=(
            jax.ShapeDtypeStruct((NROWS, H), jnp.float32),
            jax.ShapeDtypeStruct((T,), jnp.int32),
            jax.ShapeDtypeStruct((T,), jnp.int32),
            jax.ShapeDtypeStruct((T,), jnp.float32),
            jax.ShapeDtypeStruct((T,), jnp.float32),
            jax.ShapeDtypeStruct((NBPAD,), jnp.int32),
        ),
        mesh=mesh,
        compiler_params=pltpu.CompilerParams(needs_layout_passes=False),
        scratch_types=(
            pltpu.VMEM((E, T), jnp.float32),
            pltpu.VMEM((TPW, H), jnp.float32),
            pltpu.VMEM((TPW,), jnp.int32),
            pltpu.VMEM((TPW,), jnp.int32),
            pltpu.VMEM((TPW,), jnp.float32),
            pltpu.VMEM((TPW,), jnp.float32),
            pltpu.VMEM((TPW,), jnp.int32),
            pltpu.VMEM((TPW,), jnp.int32),
            pltpu.VMEM((L,), jnp.int32),
            pltpu.VMEM((L,), jnp.int32),
            pltpu.VMEM((NBPAD,), jnp.int32),
            pltpu.SemaphoreType.DMA,
        ),
    )(logitsT, x)


def _gmm_body(be_ref, xs_ref, w1_ref, w3_ref, w2_ref, y_ref):
    b = pl.program_id(0)

    @pl.when(b < be_ref[NB])
    def _():
        _gmm_compute(xs_ref, w1_ref, w3_ref, w2_ref, y_ref)


def _gmm_compute(xs_ref, w1_ref, w3_ref, w2_ref, y_ref):
    xb = xs_ref[...].astype(jnp.bfloat16)
    g = lax.dot_general(xb, w1_ref[0], (((1,), (1,)), ((), ())),
                        preferred_element_type=jnp.float32)
    u = lax.dot_general(xb, w3_ref[0], (((1,), (1,)), ((), ())),
                        preferred_element_type=jnp.float32)
    h = (g * jax.nn.sigmoid(g) * u).astype(jnp.bfloat16)
    y_ref[...] = lax.dot_general(h, w2_ref[0], (((1,), (1,)), ((), ())),
                                 preferred_element_type=jnp.float32)


def _gmm(be, xs, w1b, w3b, w2b):
    grid_spec = pltpu.PrefetchScalarGridSpec(
        num_scalar_prefetch=1,
        grid=(NB,),
        in_specs=[
            pl.BlockSpec((BT, H), lambda b, be_ref: (b, 0)),
            pl.BlockSpec((1, I, H), lambda b, be_ref: (be_ref[b], 0, 0)),
            pl.BlockSpec((1, I, H), lambda b, be_ref: (be_ref[b], 0, 0)),
            pl.BlockSpec((1, H, I), lambda b, be_ref: (be_ref[b], 0, 0)),
        ],
        out_specs=pl.BlockSpec((BT, H), lambda b, be_ref: (b, 0)),
    )
    return pl.pallas_call(
        _gmm_body,
        grid_spec=grid_spec,
        out_shape=jax.ShapeDtypeStruct((NROWS, H), jnp.float32),
    )(be, xs, w1b, w3b, w2b)


def _combine_body(y, pos0, pos1, w0, w1, out,
                  i0_v, i1_v, wv0, wv1, r0_v, r1_v, ob_v, sem):
    wid = lax.axis_index("s") * 2 + lax.axis_index("c")
    for c in range(TPW // L):
        base = wid * TPW + c * L
        pltpu.sync_copy(pos0.at[pl.ds(base, L)], i0_v)
        pltpu.sync_copy(pos1.at[pl.ds(base, L)], i1_v)
        pltpu.sync_copy(w0.at[pl.ds(base, L)], wv0)
        pltpu.sync_copy(w1.at[pl.ds(base, L)], wv1)
        g0 = pltpu.async_copy(y.at[i0_v], r0_v, sem)
        g1 = pltpu.async_copy(y.at[i1_v], r1_v, sem)
        g0.wait()
        g1.wait()

        def row(j, _):
            ws0 = plsc.load_gather(wv0, [jnp.full((L,), j, jnp.int32)])
            ws1 = plsc.load_gather(wv1, [jnp.full((L,), j, jnp.int32)])
            for q in range(H // L):
                s = pl.ds(q * L, L)
                ob_v[j, s] = ws0 * r0_v[j, s] + ws1 * r1_v[j, s]
            return 0

        lax.fori_loop(0, L, row, 0)
        pltpu.sync_copy(ob_v, out.at[pl.ds(base, L)])


def _combine(y, pos0, pos1, w0, w1):
    mesh = plsc.VectorSubcoreMesh(core_axis_name="c", subcore_axis_name="s",
                                  num_cores=2, num_subcores=16)
    return pl.kernel(
        _combine_body,
        out_type=jax.ShapeDtypeStruct((T, H), jnp.float32),
        mesh=mesh,
        compiler_params=pltpu.CompilerParams(needs_layout_passes=False),
        scratch_types=(
            pltpu.VMEM((L,), jnp.int32),
            pltpu.VMEM((L,), jnp.int32),
            pltpu.VMEM((L,), jnp.float32),
            pltpu.VMEM((L,), jnp.float32),
            pltpu.VMEM((L, H), jnp.float32),
            pltpu.VMEM((L, H), jnp.float32),
            pltpu.VMEM((L, H), jnp.float32),
            pltpu.SemaphoreType.DMA,
        ),
    )(y, pos0, pos1, w0, w1)


def kernel(hidden_states, router_logits, w1, w3, w2):
    x = hidden_states.reshape(-1, H).astype(jnp.float32)
    logitsT = router_logits.astype(jnp.float32).T
    xs, pos0, pos1, wt0, wt1, be = _dispatch(logitsT, x)
    y = _gmm(be, xs,
             w1.astype(jnp.bfloat16), w3.astype(jnp.bfloat16),
             w2.astype(jnp.bfloat16))
    return _combine(y, pos0, pos1, wt0, wt1)
